# baseline (speedup 1.0000x reference)
import functools

import jax
import jax.numpy as jnp
from jax import lax
from jax.experimental import pallas as pl
from jax.experimental.pallas import tpu as pltpu
from jax.experimental.pallas import tpu_sc as plsc


def _index_body(src_ref, dst_ref, idx_ref, eidx_in_ref, times_ref,
                src_sel_ref, y_node_ref, eidx_ref, t_ref):
    src = src_ref[...]
    dst = dst_ref[...]
    index = idx_ref[...]
    eidx_in = eidx_in_ref[...]
    times = times_ref[...]

    b = src.shape[1]
    iota = lax.broadcasted_iota(jnp.int32, (1, b), 1)

    eq_i = index == iota
    neg1 = jnp.int32(-1)
    src_sel = jnp.max(jnp.where(eq_i, src, neg1), axis=1, keepdims=True)

    ls = jnp.max(jnp.where(src_sel == src, iota, neg1), axis=1, keepdims=True)
    ld = jnp.max(jnp.where(src_sel == dst, iota, neg1), axis=1, keepdims=True)
    ld_shift = jnp.where(ld >= 0, ld + b, neg1)
    p = jnp.maximum(ls, ld_shift)
    swap = p >= b
    q = jnp.where(swap, p - b, p)

    eq_q = q == iota
    y_from_d = jnp.max(jnp.where(eq_q, dst, neg1), axis=1, keepdims=True)
    y_from_s = jnp.max(jnp.where(eq_q, src, neg1), axis=1, keepdims=True)
    y_node = jnp.where(swap, y_from_s, y_from_d)
    eidx = jnp.max(jnp.where(eq_q, eidx_in, neg1), axis=1, keepdims=True)
    t = jnp.max(jnp.where(eq_q, times, jnp.float32(-1.0)), axis=1,
                keepdims=True)

    src_sel_ref[...] = src_sel
    y_node_ref[...] = y_node
    eidx_ref[...] = eidx
    t_ref[...] = t


def _resolve_indices(source_nodes, destination_nodes, index, edge_idxs,
                     edge_times):
    b = source_nodes.shape[0]
    bsel = index.shape[0]
    nblk = 4
    blk = bsel // nblk
    full = pl.BlockSpec((1, b), lambda i: (0, 0))
    col = pl.BlockSpec((blk, 1), lambda i: (i, 0))
    out = pl.pallas_call(
        _index_body,
        grid=(nblk,),
        in_specs=[full, full, col, full, full],
        out_specs=[col, col, col, col],
        out_shape=[
            jax.ShapeDtypeStruct((bsel, 1), jnp.int32),
            jax.ShapeDtypeStruct((bsel, 1), jnp.int32),
            jax.ShapeDtypeStruct((bsel, 1), jnp.int32),
            jax.ShapeDtypeStruct((bsel, 1), jnp.float32),
        ],
    )(source_nodes.reshape(1, b), destination_nodes.reshape(1, b),
      index.reshape(bsel, 1), edge_idxs.reshape(1, b),
      edge_times.reshape(1, b))
    return out


@functools.cache
def _make_sc_gather(n_nodes, n_edges, bsel):
    info = plsc.get_sparse_core_info()
    nc, ns = info.num_cores, info.num_subcores
    nw = nc * ns
    bpw = bsel // nw
    mesh = plsc.VectorSubcoreMesh(core_axis_name="c", subcore_axis_name="s")

    @functools.partial(
        pl.kernel,
        out_type=[
            jax.ShapeDtypeStruct((bsel,), jnp.int32),
            jax.ShapeDtypeStruct((bsel,), jnp.float32),
        ],
        mesh=mesh,
        scratch_types=[
            pltpu.VMEM((bpw,), jnp.int32),
            pltpu.VMEM((bpw,), jnp.int32),
            pltpu.VMEM((bpw,), jnp.int32),
            pltpu.VMEM((bpw,), jnp.float32),
            pltpu.SemaphoreType.DMA,
        ],
    )
    def sc_gather(n2c_hbm, ef_hbm, sidx_hbm, eidx_hbm,
                  cn_out, ef_out,
                  sidx_v, eidx_v, cn_v, ef_v, sem):
        wid = lax.axis_index("s") * nc + lax.axis_index("c")
        base = wid * bpw
        pltpu.sync_copy(sidx_hbm.at[pl.ds(base, bpw)], sidx_v)
        pltpu.sync_copy(eidx_hbm.at[pl.ds(base, bpw)], eidx_v)
        cps = [
            pltpu.async_copy(n2c_hbm.at[sidx_v], cn_v, sem),
            pltpu.async_copy(ef_hbm.at[eidx_v], ef_v, sem),
        ]
        for cp in cps:
            cp.wait()
        pltpu.sync_copy(cn_v, cn_out.at[pl.ds(base, bpw)])
        pltpu.sync_copy(ef_v, ef_out.at[pl.ds(base, bpw)])

    return sc_gather


def _row_gather_body(sidx_ref, yidx_ref, mem_ref, comm_ref,
                     x_ref, y_ref, c_ref, sem):
    n = x_ref.shape[0]
    u = 8
    n_it = n // u

    def issue(it, _):
        base = it * u
        for k in range(u):
            i = base + k
            s = sidx_ref[i]
            yn = yidx_ref[i]
            pltpu.make_async_copy(mem_ref.at[pl.ds(s, 1)],
                                  x_ref.at[pl.ds(i, 1)], sem).start()
            pltpu.make_async_copy(comm_ref.at[pl.ds(s, 1)],
                                  c_ref.at[pl.ds(i, 1)], sem).start()
            pltpu.make_async_copy(mem_ref.at[pl.ds(yn, 1)],
                                  y_ref.at[pl.ds(i, 1)], sem).start()

        @pl.when(it >= 1)
        def _():
            pltpu.make_async_copy(mem_ref.at[pl.ds(0, 3 * u)],
                                  x_ref.at[pl.ds(0, 3 * u)], sem).wait()
        return 0

    lax.fori_loop(0, n_it, issue, 0, unroll=False)
    pltpu.make_async_copy(mem_ref.at[pl.ds(0, 3 * u)],
                          x_ref.at[pl.ds(0, 3 * u)], sem).wait()


def _row_gather(memory, community_embeddings, sidx, yidx):
    bsel = sidx.shape[0]
    d = memory.shape[1]
    return pl.pallas_call(
        _row_gather_body,
        in_specs=[
            pl.BlockSpec(memory_space=pltpu.MemorySpace.SMEM),
            pl.BlockSpec(memory_space=pltpu.MemorySpace.SMEM),
            pl.BlockSpec(memory_space=pl.ANY),
            pl.BlockSpec(memory_space=pl.ANY),
        ],
        out_specs=[
            pl.BlockSpec(memory_space=pltpu.MemorySpace.VMEM),
            pl.BlockSpec(memory_space=pltpu.MemorySpace.VMEM),
            pl.BlockSpec(memory_space=pltpu.MemorySpace.VMEM),
        ],
        out_shape=[
            jax.ShapeDtypeStruct((bsel, d), jnp.float32),
            jax.ShapeDtypeStruct((bsel, d), jnp.float32),
            jax.ShapeDtypeStruct((bsel, d), jnp.float32),
        ],
        scratch_shapes=[pltpu.SemaphoreType.DMA],
    )(sidx, yidx, memory, community_embeddings)


def _dense_body(x_ref, y_ref, c_ref, t_ref, ef_ref, cn_ref, ci_ref,
                tw_ref, tb_ref,
                wza_ref, wzb_ref, wze_ref, wzt_ref, uz_ref, bz_ref,
                wra_ref, wrb_ref, wre_ref, wrt_ref, ur_ref, br_ref,
                wha_ref, whb_ref, whe_ref, wht_ref, uh_ref, bh_ref,
                w1h_ref, w1c_ref, b1_ref, w2_ref, b2_ref,
                out_ref):
    x = x_ref[...]
    y = y_ref[...]
    c = c_ref[...]
    t = t_ref[...]
    ef = ef_ref[...]
    cn = cn_ref[...]
    ci = ci_ref[...]

    def dot(a, b):
        return lax.dot_general(a, b, (((1,), (0,)), ((), ())),
                               precision=lax.Precision.HIGHEST,
                               preferred_element_type=jnp.float32)

    tenc = jnp.cos(t * tw_ref[...] + tb_ref[...])

    pre_z = (dot(x, wza_ref[...] + uz_ref[...]) + dot(y, wzb_ref[...])
             + dot(tenc, wzt_ref[...]) + ef * wze_ref[...] + bz_ref[...])
    z = jax.nn.sigmoid(pre_z)
    pre_r = (dot(x, wra_ref[...] + ur_ref[...]) + dot(y, wrb_ref[...])
             + dot(tenc, wrt_ref[...]) + ef * wre_ref[...] + br_ref[...])
    r = jax.nn.sigmoid(pre_r)
    pre_n = (dot(x, wha_ref[...]) + dot(y, whb_ref[...])
             + dot(tenc, wht_ref[...]) + ef * whe_ref[...] + bh_ref[...]
             + dot(r * x, uh_ref[...]))
    n = jnp.tanh(pre_n)
    h = (1.0 - z) * n + z * x

    h1 = jax.nn.relu(dot(h, w1h_ref[...]) + dot(c, w1c_ref[...]) + b1_ref[...])
    o = jnp.sum(h1 * w2_ref[...], axis=1, keepdims=True) + b2_ref[...]
    pred = jax.nn.sigmoid(o)

    member = jnp.max((cn == ci).astype(jnp.float32), axis=1, keepdims=True)
    out_ref[...] = pred * member


def kernel(source_nodes, destination_nodes, edge_times, edge_idxs, index,
           memory, community_embeddings, node2community, community_index,
           edge_features, time_w, time_b,
           gru_Wz, gru_Uz, gru_bz, gru_Wr, gru_Ur, gru_br,
           gru_Wh, gru_Uh, gru_bh,
           pred_W1, pred_b1, pred_W2, pred_b2):
    b = source_nodes.shape[0]
    bsel = index.shape[0]
    n_nodes, d = memory.shape
    n_edges = edge_features.shape[0]

    src_sel, y_node, eidx, t = _resolve_indices(
        source_nodes, destination_nodes, index, edge_idxs, edge_times)

    sc_gather = _make_sc_gather(n_nodes, n_edges, bsel)
    cn, ef = sc_gather(
        node2community, edge_features.reshape(n_edges),
        src_sel.reshape(bsel), eidx.reshape(bsel))
    x, y, c = _row_gather(memory, community_embeddings,
                          src_sel.reshape(bsel), y_node.reshape(bsel))

    ci_pad = jnp.full((1, bsel), -1, dtype=jnp.int32)
    ci_pad = lax.dynamic_update_slice(
        ci_pad, community_index.astype(jnp.int32).reshape(1, -1), (0, 0))

    def parts(w):
        return (w[:d], w[d:2 * d], w[2 * d:2 * d + 1], w[2 * d + 1:])

    wza, wzb, wze, wzt = parts(gru_Wz)
    wra, wrb, wre, wrt = parts(gru_Wr)
    wha, whb, whe, wht = parts(gru_Wh)

    pred = pl.pallas_call(
        _dense_body,
        out_shape=jax.ShapeDtypeStruct((bsel, 1), jnp.float32),
    )(x, y, c, t, ef.reshape(bsel, 1), cn.reshape(bsel, 1), ci_pad,
      time_w.reshape(1, d), time_b.reshape(1, d),
      wza, wzb, wze, wzt, gru_Uz, gru_bz.reshape(1, d),
      wra, wrb, wre, wrt, gru_Ur, gru_br.reshape(1, d),
      wha, whb, whe, wht, gru_Uh, gru_bh.reshape(1, d),
      pred_W1[:d], pred_W1[d:], pred_b1.reshape(1, d),
      pred_W2.reshape(1, d), pred_b2.reshape(1, 1))
    return pred


# speedup vs baseline: 1.0820x; 1.0820x over previous
import functools

import jax
import jax.numpy as jnp
from jax import lax
from jax.experimental import pallas as pl
from jax.experimental.pallas import tpu as pltpu
from jax.experimental.pallas import tpu_sc as plsc


def _index_body(src_ref, dst_ref, idx_ref, eidx_in_ref, times_ref,
                src_sel_ref, y_node_ref, eidx_ref, t_ref):
    src = src_ref[...]
    dst = dst_ref[...]
    index = idx_ref[...]
    eidx_in = eidx_in_ref[...]
    times = times_ref[...]

    b = src.shape[1]
    iota = lax.broadcasted_iota(jnp.int32, (1, b), 1)

    eq_i = index == iota
    neg1 = jnp.int32(-1)
    src_sel = jnp.max(jnp.where(eq_i, src, neg1), axis=1, keepdims=True)

    ls = jnp.max(jnp.where(src_sel == src, iota, neg1), axis=1, keepdims=True)
    ld = jnp.max(jnp.where(src_sel == dst, iota, neg1), axis=1, keepdims=True)
    ld_shift = jnp.where(ld >= 0, ld + b, neg1)
    p = jnp.maximum(ls, ld_shift)
    swap = p >= b
    q = jnp.where(swap, p - b, p)

    eq_q = q == iota
    y_from_d = jnp.max(jnp.where(eq_q, dst, neg1), axis=1, keepdims=True)
    y_from_s = jnp.max(jnp.where(eq_q, src, neg1), axis=1, keepdims=True)
    y_node = jnp.where(swap, y_from_s, y_from_d)
    eidx = jnp.max(jnp.where(eq_q, eidx_in, neg1), axis=1, keepdims=True)
    t = jnp.max(jnp.where(eq_q, times, jnp.float32(-1.0)), axis=1,
                keepdims=True)

    src_sel_ref[...] = src_sel
    y_node_ref[...] = y_node
    eidx_ref[...] = eidx
    t_ref[...] = t


def _resolve_indices(source_nodes, destination_nodes, index, edge_idxs,
                     edge_times):
    b = source_nodes.shape[0]
    bsel = index.shape[0]
    nblk = 4
    blk = bsel // nblk
    full = pl.BlockSpec((1, b), lambda i: (0, 0))
    col = pl.BlockSpec((blk, 1), lambda i: (i, 0))
    out = pl.pallas_call(
        _index_body,
        grid=(nblk,),
        in_specs=[full, full, col, full, full],
        out_specs=[col, col, col, col],
        out_shape=[
            jax.ShapeDtypeStruct((bsel, 1), jnp.int32),
            jax.ShapeDtypeStruct((bsel, 1), jnp.int32),
            jax.ShapeDtypeStruct((bsel, 1), jnp.int32),
            jax.ShapeDtypeStruct((bsel, 1), jnp.float32),
        ],
    )(source_nodes.reshape(1, b), destination_nodes.reshape(1, b),
      index.reshape(bsel, 1), edge_idxs.reshape(1, b),
      edge_times.reshape(1, b))
    return out


@functools.cache
def _make_sc_gather(n_nodes, n_edges, bsel):
    info = plsc.get_sparse_core_info()
    nc, ns = info.num_cores, info.num_subcores
    nw = nc * ns
    bpw = bsel // nw
    mesh = plsc.VectorSubcoreMesh(core_axis_name="c", subcore_axis_name="s")

    @functools.partial(
        pl.kernel,
        out_type=[
            jax.ShapeDtypeStruct((bsel,), jnp.int32),
            jax.ShapeDtypeStruct((bsel,), jnp.float32),
        ],
        mesh=mesh,
        scratch_types=[
            pltpu.VMEM((bpw,), jnp.int32),
            pltpu.VMEM((bpw,), jnp.int32),
            pltpu.VMEM((bpw,), jnp.int32),
            pltpu.VMEM((bpw,), jnp.float32),
            pltpu.SemaphoreType.DMA,
        ],
    )
    def sc_gather(n2c_hbm, ef_hbm, sidx_hbm, eidx_hbm,
                  cn_out, ef_out,
                  sidx_v, eidx_v, cn_v, ef_v, sem):
        wid = lax.axis_index("s") * nc + lax.axis_index("c")
        base = wid * bpw
        pltpu.sync_copy(sidx_hbm.at[pl.ds(base, bpw)], sidx_v)
        pltpu.sync_copy(eidx_hbm.at[pl.ds(base, bpw)], eidx_v)
        cps = [
            pltpu.async_copy(n2c_hbm.at[sidx_v], cn_v, sem),
            pltpu.async_copy(ef_hbm.at[eidx_v], ef_v, sem),
        ]
        for cp in cps:
            cp.wait()
        pltpu.sync_copy(cn_v, cn_out.at[pl.ds(base, bpw)])
        pltpu.sync_copy(ef_v, ef_out.at[pl.ds(base, bpw)])

    return sc_gather


def _row_gather_body(sidx_ref, yidx_ref, mem_ref, comm_ref,
                     x_ref, y_ref, c_ref, sem):
    n = x_ref.shape[0]
    u = 8
    n_it = n // u

    def issue(it, _):
        base = it * u
        for k in range(u):
            i = base + k
            s = sidx_ref[i]
            yn = yidx_ref[i]
            pltpu.make_async_copy(mem_ref.at[pl.ds(s, 1)],
                                  x_ref.at[pl.ds(i, 1)], sem).start()
            pltpu.make_async_copy(comm_ref.at[pl.ds(s, 1)],
                                  c_ref.at[pl.ds(i, 1)], sem).start()
            pltpu.make_async_copy(mem_ref.at[pl.ds(yn, 1)],
                                  y_ref.at[pl.ds(i, 1)], sem).start()

        @pl.when(it >= lag_it)
        def _():
            pltpu.make_async_copy(mem_ref.at[pl.ds(0, 3 * u)],
                                  x_ref.at[pl.ds(0, 3 * u)], sem).wait()
        return 0

    lag_it = 8
    lax.fori_loop(0, n_it, issue, 0, unroll=False)
    for _ in range(lag_it):
        pltpu.make_async_copy(mem_ref.at[pl.ds(0, 3 * u)],
                              x_ref.at[pl.ds(0, 3 * u)], sem).wait()


def _row_gather(memory, community_embeddings, sidx, yidx):
    bsel = sidx.shape[0]
    d = memory.shape[1]
    return pl.pallas_call(
        _row_gather_body,
        in_specs=[
            pl.BlockSpec(memory_space=pltpu.MemorySpace.SMEM),
            pl.BlockSpec(memory_space=pltpu.MemorySpace.SMEM),
            pl.BlockSpec(memory_space=pl.ANY),
            pl.BlockSpec(memory_space=pl.ANY),
        ],
        out_specs=[
            pl.BlockSpec(memory_space=pltpu.MemorySpace.VMEM),
            pl.BlockSpec(memory_space=pltpu.MemorySpace.VMEM),
            pl.BlockSpec(memory_space=pltpu.MemorySpace.VMEM),
        ],
        out_shape=[
            jax.ShapeDtypeStruct((bsel, d), jnp.float32),
            jax.ShapeDtypeStruct((bsel, d), jnp.float32),
            jax.ShapeDtypeStruct((bsel, d), jnp.float32),
        ],
        scratch_shapes=[pltpu.SemaphoreType.DMA],
    )(sidx, yidx, memory, community_embeddings)


def _dense_body(x_ref, y_ref, c_ref, t_ref, ef_ref, cn_ref, ci_ref,
                tw_ref, tb_ref,
                wza_ref, wzb_ref, wze_ref, wzt_ref, uz_ref, bz_ref,
                wra_ref, wrb_ref, wre_ref, wrt_ref, ur_ref, br_ref,
                wha_ref, whb_ref, whe_ref, wht_ref, uh_ref, bh_ref,
                w1h_ref, w1c_ref, b1_ref, w2_ref, b2_ref,
                out_ref):
    x = x_ref[...]
    y = y_ref[...]
    c = c_ref[...]
    t = t_ref[...]
    ef = ef_ref[...]
    cn = cn_ref[...]
    ci = ci_ref[...]

    def dot(a, b):
        return lax.dot_general(a, b, (((1,), (0,)), ((), ())),
                               precision=lax.Precision.HIGHEST,
                               preferred_element_type=jnp.float32)

    tenc = jnp.cos(t * tw_ref[...] + tb_ref[...])

    pre_z = (dot(x, wza_ref[...] + uz_ref[...]) + dot(y, wzb_ref[...])
             + dot(tenc, wzt_ref[...]) + ef * wze_ref[...] + bz_ref[...])
    z = jax.nn.sigmoid(pre_z)
    pre_r = (dot(x, wra_ref[...] + ur_ref[...]) + dot(y, wrb_ref[...])
             + dot(tenc, wrt_ref[...]) + ef * wre_ref[...] + br_ref[...])
    r = jax.nn.sigmoid(pre_r)
    pre_n = (dot(x, wha_ref[...]) + dot(y, whb_ref[...])
             + dot(tenc, wht_ref[...]) + ef * whe_ref[...] + bh_ref[...]
             + dot(r * x, uh_ref[...]))
    n = jnp.tanh(pre_n)
    h = (1.0 - z) * n + z * x

    h1 = jax.nn.relu(dot(h, w1h_ref[...]) + dot(c, w1c_ref[...]) + b1_ref[...])
    o = jnp.sum(h1 * w2_ref[...], axis=1, keepdims=True) + b2_ref[...]
    pred = jax.nn.sigmoid(o)

    member = jnp.max((cn == ci).astype(jnp.float32), axis=1, keepdims=True)
    out_ref[...] = pred * member


def kernel(source_nodes, destination_nodes, edge_times, edge_idxs, index,
           memory, community_embeddings, node2community, community_index,
           edge_features, time_w, time_b,
           gru_Wz, gru_Uz, gru_bz, gru_Wr, gru_Ur, gru_br,
           gru_Wh, gru_Uh, gru_bh,
           pred_W1, pred_b1, pred_W2, pred_b2):
    b = source_nodes.shape[0]
    bsel = index.shape[0]
    n_nodes, d = memory.shape
    n_edges = edge_features.shape[0]

    src_sel, y_node, eidx, t = _resolve_indices(
        source_nodes, destination_nodes, index, edge_idxs, edge_times)

    sc_gather = _make_sc_gather(n_nodes, n_edges, bsel)
    cn, ef = sc_gather(
        node2community, edge_features.reshape(n_edges),
        src_sel.reshape(bsel), eidx.reshape(bsel))
    x, y, c = _row_gather(memory, community_embeddings,
                          src_sel.reshape(bsel), y_node.reshape(bsel))

    ci_pad = jnp.full((1, bsel), -1, dtype=jnp.int32)
    ci_pad = lax.dynamic_update_slice(
        ci_pad, community_index.astype(jnp.int32).reshape(1, -1), (0, 0))

    def parts(w):
        return (w[:d], w[d:2 * d], w[2 * d:2 * d + 1], w[2 * d + 1:])

    wza, wzb, wze, wzt = parts(gru_Wz)
    wra, wrb, wre, wrt = parts(gru_Wr)
    wha, whb, whe, wht = parts(gru_Wh)

    pred = pl.pallas_call(
        _dense_body,
        out_shape=jax.ShapeDtypeStruct((bsel, 1), jnp.float32),
    )(x, y, c, t, ef.reshape(bsel, 1), cn.reshape(bsel, 1), ci_pad,
      time_w.reshape(1, d), time_b.reshape(1, d),
      wza, wzb, wze, wzt, gru_Uz, gru_bz.reshape(1, d),
      wra, wrb, wre, wrt, gru_Ur, gru_br.reshape(1, d),
      wha, whb, whe, wht, gru_Uh, gru_bh.reshape(1, d),
      pred_W1[:d], pred_W1[d:], pred_b1.reshape(1, d),
      pred_W2.reshape(1, d), pred_b2.reshape(1, 1))
    return pred


# speedup vs baseline: 1.0928x; 1.0100x over previous
import functools

import jax
import jax.numpy as jnp
from jax import lax
from jax.experimental import pallas as pl
from jax.experimental.pallas import tpu as pltpu
from jax.experimental.pallas import tpu_sc as plsc


def _index_body(src_ref, dst_ref, idx_ref, eidx_in_ref, times_ref,
                src_sel_ref, y_node_ref, eidx_ref, t_ref):
    src = src_ref[...]
    dst = dst_ref[...]
    index = idx_ref[...]
    eidx_in = eidx_in_ref[...]
    times = times_ref[...]

    b = src.shape[1]
    iota = lax.broadcasted_iota(jnp.int32, (1, b), 1)

    eq_i = index == iota
    neg1 = jnp.int32(-1)
    src_sel = jnp.max(jnp.where(eq_i, src, neg1), axis=1, keepdims=True)

    ls = jnp.max(jnp.where(src_sel == src, iota, neg1), axis=1, keepdims=True)
    ld = jnp.max(jnp.where(src_sel == dst, iota, neg1), axis=1, keepdims=True)
    ld_shift = jnp.where(ld >= 0, ld + b, neg1)
    p = jnp.maximum(ls, ld_shift)
    swap = p >= b
    q = jnp.where(swap, p - b, p)

    eq_q = q == iota
    y_from_d = jnp.max(jnp.where(eq_q, dst, neg1), axis=1, keepdims=True)
    y_from_s = jnp.max(jnp.where(eq_q, src, neg1), axis=1, keepdims=True)
    y_node = jnp.where(swap, y_from_s, y_from_d)
    eidx = jnp.max(jnp.where(eq_q, eidx_in, neg1), axis=1, keepdims=True)
    t = jnp.max(jnp.where(eq_q, times, jnp.float32(-1.0)), axis=1,
                keepdims=True)

    src_sel_ref[...] = src_sel
    y_node_ref[...] = y_node
    eidx_ref[...] = eidx
    t_ref[...] = t


def _resolve_indices(source_nodes, destination_nodes, index, edge_idxs,
                     edge_times):
    b = source_nodes.shape[0]
    bsel = index.shape[0]
    nblk = 4
    blk = bsel // nblk
    full = pl.BlockSpec((1, b), lambda i: (0, 0))
    col = pl.BlockSpec((blk, 1), lambda i: (i, 0))
    out = pl.pallas_call(
        _index_body,
        grid=(nblk,),
        in_specs=[full, full, col, full, full],
        out_specs=[col, col, col, col],
        out_shape=[
            jax.ShapeDtypeStruct((bsel, 1), jnp.int32),
            jax.ShapeDtypeStruct((bsel, 1), jnp.int32),
            jax.ShapeDtypeStruct((bsel, 1), jnp.int32),
            jax.ShapeDtypeStruct((bsel, 1), jnp.float32),
        ],
    )(source_nodes.reshape(1, b), destination_nodes.reshape(1, b),
      index.reshape(bsel, 1), edge_idxs.reshape(1, b),
      edge_times.reshape(1, b))
    return out


@functools.cache
def _make_sc_gather(n_nodes, n_edges, bsel):
    info = plsc.get_sparse_core_info()
    nc, ns = info.num_cores, info.num_subcores
    nw = nc * ns
    bpw = bsel // nw
    mesh = plsc.VectorSubcoreMesh(core_axis_name="c", subcore_axis_name="s")

    @functools.partial(
        pl.kernel,
        out_type=[
            jax.ShapeDtypeStruct((bsel,), jnp.int32),
            jax.ShapeDtypeStruct((bsel,), jnp.float32),
        ],
        mesh=mesh,
        scratch_types=[
            pltpu.VMEM((bpw,), jnp.int32),
            pltpu.VMEM((bpw,), jnp.int32),
            pltpu.VMEM((bpw,), jnp.int32),
            pltpu.VMEM((bpw,), jnp.float32),
            pltpu.SemaphoreType.DMA,
        ],
    )
    def sc_gather(n2c_hbm, ef_hbm, sidx_hbm, eidx_hbm,
                  cn_out, ef_out,
                  sidx_v, eidx_v, cn_v, ef_v, sem):
        wid = lax.axis_index("s") * nc + lax.axis_index("c")
        base = wid * bpw
        pltpu.sync_copy(sidx_hbm.at[pl.ds(base, bpw)], sidx_v)
        pltpu.sync_copy(eidx_hbm.at[pl.ds(base, bpw)], eidx_v)
        cps = [
            pltpu.async_copy(n2c_hbm.at[sidx_v], cn_v, sem),
            pltpu.async_copy(ef_hbm.at[eidx_v], ef_v, sem),
        ]
        for cp in cps:
            cp.wait()
        pltpu.sync_copy(cn_v, cn_out.at[pl.ds(base, bpw)])
        pltpu.sync_copy(ef_v, ef_out.at[pl.ds(base, bpw)])

    return sc_gather


def _fused_body(sidx_ref, yidx_ref, t_ref, ef_ref, cn_ref, ci_ref,
                mem_ref, comm_ref,
                tw_ref, tb_ref,
                wza_ref, wzb_ref, wze_ref, wzt_ref, uz_ref, bz_ref,
                wra_ref, wrb_ref, wre_ref, wrt_ref, ur_ref, br_ref,
                wha_ref, whb_ref, whe_ref, wht_ref, uh_ref, bh_ref,
                w1h_ref, w1c_ref, b1_ref, w2_ref, b2_ref,
                out_ref,
                xb_ref, yb_ref, cb_ref, sem0, sem1):
    k = pl.program_id(0)
    nblk = pl.num_programs(0)
    blk = out_ref.shape[0]

    def issue(blk_id, slot, sem):
        u = 8

        def one(it, _):
            base = blk_id * blk + it * u
            for kk in range(u):
                i = base + kk
                loc = it * u + kk
                s = sidx_ref[i]
                yn = yidx_ref[i]
                pltpu.make_async_copy(
                    mem_ref.at[pl.ds(s, 1)],
                    xb_ref.at[slot, pl.ds(loc, 1)], sem).start()
                pltpu.make_async_copy(
                    comm_ref.at[pl.ds(s, 1)],
                    cb_ref.at[slot, pl.ds(loc, 1)], sem).start()
                pltpu.make_async_copy(
                    mem_ref.at[pl.ds(yn, 1)],
                    yb_ref.at[slot, pl.ds(loc, 1)], sem).start()
            return 0

        lax.fori_loop(0, blk // u, one, 0, unroll=False)

    def wait_block(sem):
        for _ in range(3):
            pltpu.make_async_copy(mem_ref.at[pl.ds(0, blk)],
                                  xb_ref.at[0], sem).wait()

    @pl.when(k == 0)
    def _():
        issue(jnp.int32(0), 0, sem0)
        issue(jnp.int32(1), 1, sem1)

    @pl.when((k >= 1) & (k + 1 < nblk))
    def _():
        nxt = k + 1

        @pl.when(lax.rem(nxt, 2) == 0)
        def _():
            issue(nxt, 0, sem0)

        @pl.when(lax.rem(nxt, 2) == 1)
        def _():
            issue(nxt, 1, sem1)

    @pl.when(lax.rem(k, 2) == 0)
    def _():
        wait_block(sem0)

    @pl.when(lax.rem(k, 2) == 1)
    def _():
        wait_block(sem1)

    odd = (lax.rem(k, 2) == 1)
    x = jnp.where(odd, xb_ref[1], xb_ref[0])
    y = jnp.where(odd, yb_ref[1], yb_ref[0])
    c = jnp.where(odd, cb_ref[1], cb_ref[0])
    t = t_ref[...]
    ef = ef_ref[...]
    cn = cn_ref[...]
    ci = ci_ref[...]

    def dot(a, b):
        return lax.dot_general(a, b, (((1,), (0,)), ((), ())),
                               precision=lax.Precision.HIGHEST,
                               preferred_element_type=jnp.float32)

    tenc = jnp.cos(t * tw_ref[...] + tb_ref[...])

    pre_z = (dot(x, wza_ref[...] + uz_ref[...]) + dot(y, wzb_ref[...])
             + dot(tenc, wzt_ref[...]) + ef * wze_ref[...] + bz_ref[...])
    z = jax.nn.sigmoid(pre_z)
    pre_r = (dot(x, wra_ref[...] + ur_ref[...]) + dot(y, wrb_ref[...])
             + dot(tenc, wrt_ref[...]) + ef * wre_ref[...] + br_ref[...])
    r = jax.nn.sigmoid(pre_r)
    pre_n = (dot(x, wha_ref[...]) + dot(y, whb_ref[...])
             + dot(tenc, wht_ref[...]) + ef * whe_ref[...] + bh_ref[...]
             + dot(r * x, uh_ref[...]))
    n = jnp.tanh(pre_n)
    h = (1.0 - z) * n + z * x

    h1 = jax.nn.relu(dot(h, w1h_ref[...]) + dot(c, w1c_ref[...]) + b1_ref[...])
    o = jnp.sum(h1 * w2_ref[...], axis=1, keepdims=True) + b2_ref[...]
    pred = jax.nn.sigmoid(o)

    member = jnp.max((cn == ci).astype(jnp.float32), axis=1, keepdims=True)
    out_ref[...] = pred * member


def kernel(source_nodes, destination_nodes, edge_times, edge_idxs, index,
           memory, community_embeddings, node2community, community_index,
           edge_features, time_w, time_b,
           gru_Wz, gru_Uz, gru_bz, gru_Wr, gru_Ur, gru_br,
           gru_Wh, gru_Uh, gru_bh,
           pred_W1, pred_b1, pred_W2, pred_b2):
    b = source_nodes.shape[0]
    bsel = index.shape[0]
    n_nodes, d = memory.shape
    n_edges = edge_features.shape[0]

    src_sel, y_node, eidx, t = _resolve_indices(
        source_nodes, destination_nodes, index, edge_idxs, edge_times)

    sc_gather = _make_sc_gather(n_nodes, n_edges, bsel)
    cn, ef = sc_gather(
        node2community, edge_features.reshape(n_edges),
        src_sel.reshape(bsel), eidx.reshape(bsel))

    ci_pad = jnp.full((1, bsel), -1, dtype=jnp.int32)
    ci_pad = lax.dynamic_update_slice(
        ci_pad, community_index.astype(jnp.int32).reshape(1, -1), (0, 0))

    def parts(w):
        return (w[:d], w[d:2 * d], w[2 * d:2 * d + 1], w[2 * d + 1:])

    wza, wzb, wze, wzt = parts(gru_Wz)
    wra, wrb, wre, wrt = parts(gru_Wr)
    wha, whb, whe, wht = parts(gru_Wh)

    nblk = 4
    blk = bsel // nblk
    col = pl.BlockSpec((blk, 1), lambda i: (i, 0))
    full = lambda shape: pl.BlockSpec(shape, lambda i: tuple(0 for _ in shape))
    smem = pl.BlockSpec(memory_space=pltpu.MemorySpace.SMEM)
    anyspace = pl.BlockSpec(memory_space=pl.ANY)

    pred = pl.pallas_call(
        _fused_body,
        grid=(nblk,),
        in_specs=[smem, smem, col, col, col, full((1, bsel)),
                  anyspace, anyspace]
                 + [full((1, d))] * 2
                 + [full((d, d)), full((d, d)), full((1, d)), full((d, d)),
                    full((d, d)), full((1, d))] * 3
                 + [full((d, d)), full((d, d)), full((1, d)),
                    full((1, d)), full((1, 1))],
        out_specs=col,
        out_shape=jax.ShapeDtypeStruct((bsel, 1), jnp.float32),
        scratch_shapes=[
            pltpu.VMEM((2, blk, d), jnp.float32),
            pltpu.VMEM((2, blk, d), jnp.float32),
            pltpu.VMEM((2, blk, d), jnp.float32),
            pltpu.SemaphoreType.DMA,
            pltpu.SemaphoreType.DMA,
        ],
    )(src_sel.reshape(bsel), y_node.reshape(bsel),
      t, ef.reshape(bsel, 1), cn.reshape(bsel, 1), ci_pad,
      memory, community_embeddings,
      time_w.reshape(1, d), time_b.reshape(1, d),
      wza, wzb, wze.reshape(1, d), wzt, gru_Uz, gru_bz.reshape(1, d),
      wra, wrb, wre.reshape(1, d), wrt, gru_Ur, gru_br.reshape(1, d),
      wha, whb, whe.reshape(1, d), wht, gru_Uh, gru_bh.reshape(1, d),
      pred_W1[:d], pred_W1[d:], pred_b1.reshape(1, d),
      pred_W2.reshape(1, d), pred_b2.reshape(1, 1))
    return pred


# speedup vs baseline: 1.1687x; 1.0695x over previous
import functools

import jax
import jax.numpy as jnp
from jax import lax
from jax.experimental import pallas as pl
from jax.experimental.pallas import tpu as pltpu
from jax.experimental.pallas import tpu_sc as plsc


def _index_body(src_ref, dst_ref, idx_ref, eidx_in_ref, times_ref,
                src_sel_ref, y_node_ref, eidx_ref, t_ref):
    src = src_ref[...]
    dst = dst_ref[...]
    index = idx_ref[...]
    eidx_in = eidx_in_ref[...]
    times = times_ref[...]

    b = src.shape[1]
    iota = lax.broadcasted_iota(jnp.int32, (1, b), 1)

    eq_i = index == iota
    neg1 = jnp.int32(-1)
    src_sel = jnp.max(jnp.where(eq_i, src, neg1), axis=1, keepdims=True)

    ls = jnp.max(jnp.where(src_sel == src, iota, neg1), axis=1, keepdims=True)
    ld = jnp.max(jnp.where(src_sel == dst, iota, neg1), axis=1, keepdims=True)
    ld_shift = jnp.where(ld >= 0, ld + b, neg1)
    p = jnp.maximum(ls, ld_shift)
    swap = p >= b
    q = jnp.where(swap, p - b, p)

    eq_q = q == iota
    y_from_d = jnp.max(jnp.where(eq_q, dst, neg1), axis=1, keepdims=True)
    y_from_s = jnp.max(jnp.where(eq_q, src, neg1), axis=1, keepdims=True)
    y_node = jnp.where(swap, y_from_s, y_from_d)
    eidx = jnp.max(jnp.where(eq_q, eidx_in, neg1), axis=1, keepdims=True)
    t = jnp.max(jnp.where(eq_q, times, jnp.float32(-1.0)), axis=1,
                keepdims=True)

    src_sel_ref[...] = src_sel
    y_node_ref[...] = y_node
    eidx_ref[...] = eidx
    t_ref[...] = t


def _resolve_indices(source_nodes, destination_nodes, index, edge_idxs,
                     edge_times):
    b = source_nodes.shape[0]
    bsel = index.shape[0]
    nblk = 4
    blk = bsel // nblk
    full = pl.BlockSpec((1, b), lambda i: (0, 0))
    col = pl.BlockSpec((blk, 1), lambda i: (i, 0))
    out = pl.pallas_call(
        _index_body,
        grid=(nblk,),
        in_specs=[full, full, col, full, full],
        out_specs=[col, col, col, col],
        out_shape=[
            jax.ShapeDtypeStruct((bsel, 1), jnp.int32),
            jax.ShapeDtypeStruct((bsel, 1), jnp.int32),
            jax.ShapeDtypeStruct((bsel, 1), jnp.int32),
            jax.ShapeDtypeStruct((bsel, 1), jnp.float32),
        ],
    )(source_nodes.reshape(1, b), destination_nodes.reshape(1, b),
      index.reshape(bsel, 1), edge_idxs.reshape(1, b),
      edge_times.reshape(1, b))
    return out


@functools.cache
def _make_sc_gather(n_nodes, n_edges, bsel):
    info = plsc.get_sparse_core_info()
    nc, ns = info.num_cores, info.num_subcores
    nw = nc * ns
    bpw = bsel // nw
    mesh = plsc.VectorSubcoreMesh(core_axis_name="c", subcore_axis_name="s")

    @functools.partial(
        pl.kernel,
        out_type=[
            jax.ShapeDtypeStruct((bsel,), jnp.int32),
            jax.ShapeDtypeStruct((bsel,), jnp.float32),
        ],
        mesh=mesh,
        scratch_types=[
            pltpu.VMEM((bpw,), jnp.int32),
            pltpu.VMEM((bpw,), jnp.int32),
            pltpu.VMEM((bpw,), jnp.int32),
            pltpu.VMEM((bpw,), jnp.float32),
            pltpu.SemaphoreType.DMA,
        ],
    )
    def sc_gather(n2c_hbm, ef_hbm, sidx_hbm, eidx_hbm,
                  cn_out, ef_out,
                  sidx_v, eidx_v, cn_v, ef_v, sem):
        wid = lax.axis_index("s") * nc + lax.axis_index("c")
        base = wid * bpw
        pltpu.sync_copy(sidx_hbm.at[pl.ds(base, bpw)], sidx_v)
        pltpu.sync_copy(eidx_hbm.at[pl.ds(base, bpw)], eidx_v)
        cps = [
            pltpu.async_copy(n2c_hbm.at[sidx_v], cn_v, sem),
            pltpu.async_copy(ef_hbm.at[eidx_v], ef_v, sem),
        ]
        for cp in cps:
            cp.wait()
        pltpu.sync_copy(cn_v, cn_out.at[pl.ds(base, bpw)])
        pltpu.sync_copy(ef_v, ef_out.at[pl.ds(base, bpw)])

    return sc_gather


def _fused_body(sidx_ref, yidx_ref, t_ref, ef_ref, cn_ref, ci_ref,
                mem_ref, comm_ref,
                tw_ref, tb_ref,
                wza_ref, wzb_ref, wze_ref, wzt_ref, uz_ref, bz_ref,
                wra_ref, wrb_ref, wre_ref, wrt_ref, ur_ref, br_ref,
                wha_ref, whb_ref, whe_ref, wht_ref, uh_ref, bh_ref,
                w1h_ref, w1c_ref, b1_ref, w2_ref, b2_ref,
                out_ref,
                xb_ref, yb_ref, cb_ref, sem0, sem1):
    k = pl.program_id(0)
    nblk = pl.num_programs(0)
    blk = out_ref.shape[0]

    def issue(blk_id, slot, sem):
        u = 8

        def one(it, _):
            base = blk_id * blk + it * u
            for kk in range(u):
                i = base + kk
                loc = it * u + kk
                s = sidx_ref[i]
                yn = yidx_ref[i]
                pltpu.make_async_copy(
                    mem_ref.at[pl.ds(s, 1)],
                    xb_ref.at[slot, pl.ds(loc, 1)], sem).start()
                pltpu.make_async_copy(
                    comm_ref.at[pl.ds(s, 1)],
                    cb_ref.at[slot, pl.ds(loc, 1)], sem).start()
                pltpu.make_async_copy(
                    mem_ref.at[pl.ds(yn, 1)],
                    yb_ref.at[slot, pl.ds(loc, 1)], sem).start()
            return 0

        lax.fori_loop(0, blk // u, one, 0, unroll=False)

    def wait_block(sem):
        for _ in range(3):
            pltpu.make_async_copy(mem_ref.at[pl.ds(0, blk)],
                                  xb_ref.at[0], sem).wait()

    @pl.when(k == 0)
    def _():
        issue(jnp.int32(0), 0, sem0)
        issue(jnp.int32(1), 1, sem1)

    @pl.when((k >= 1) & (k + 1 < nblk))
    def _():
        nxt = k + 1

        @pl.when(lax.rem(nxt, 2) == 0)
        def _():
            issue(nxt, 0, sem0)

        @pl.when(lax.rem(nxt, 2) == 1)
        def _():
            issue(nxt, 1, sem1)

    @pl.when(lax.rem(k, 2) == 0)
    def _():
        wait_block(sem0)

    @pl.when(lax.rem(k, 2) == 1)
    def _():
        wait_block(sem1)

    odd = (lax.rem(k, 2) == 1)
    x = jnp.where(odd, xb_ref[1], xb_ref[0])
    y = jnp.where(odd, yb_ref[1], yb_ref[0])
    c = jnp.where(odd, cb_ref[1], cb_ref[0])
    t = t_ref[...]
    ef = ef_ref[...]
    cn = cn_ref[...]
    ci = ci_ref[...]

    def dot(a, b):
        return lax.dot_general(a, b, (((1,), (0,)), ((), ())),
                               precision=lax.Precision.DEFAULT,
                               preferred_element_type=jnp.float32)

    tenc = jnp.cos(t * tw_ref[...] + tb_ref[...])

    pre_z = (dot(x, wza_ref[...] + uz_ref[...]) + dot(y, wzb_ref[...])
             + dot(tenc, wzt_ref[...]) + ef * wze_ref[...] + bz_ref[...])
    z = jax.nn.sigmoid(pre_z)
    pre_r = (dot(x, wra_ref[...] + ur_ref[...]) + dot(y, wrb_ref[...])
             + dot(tenc, wrt_ref[...]) + ef * wre_ref[...] + br_ref[...])
    r = jax.nn.sigmoid(pre_r)
    pre_n = (dot(x, wha_ref[...]) + dot(y, whb_ref[...])
             + dot(tenc, wht_ref[...]) + ef * whe_ref[...] + bh_ref[...]
             + dot(r * x, uh_ref[...]))
    n = jnp.tanh(pre_n)
    h = (1.0 - z) * n + z * x

    h1 = jax.nn.relu(dot(h, w1h_ref[...]) + dot(c, w1c_ref[...]) + b1_ref[...])
    o = jnp.sum(h1 * w2_ref[...], axis=1, keepdims=True) + b2_ref[...]
    pred = jax.nn.sigmoid(o)

    member = jnp.max((cn == ci).astype(jnp.float32), axis=1, keepdims=True)
    out_ref[...] = pred * member


def kernel(source_nodes, destination_nodes, edge_times, edge_idxs, index,
           memory, community_embeddings, node2community, community_index,
           edge_features, time_w, time_b,
           gru_Wz, gru_Uz, gru_bz, gru_Wr, gru_Ur, gru_br,
           gru_Wh, gru_Uh, gru_bh,
           pred_W1, pred_b1, pred_W2, pred_b2):
    b = source_nodes.shape[0]
    bsel = index.shape[0]
    n_nodes, d = memory.shape
    n_edges = edge_features.shape[0]

    src_sel, y_node, eidx, t = _resolve_indices(
        source_nodes, destination_nodes, index, edge_idxs, edge_times)

    sc_gather = _make_sc_gather(n_nodes, n_edges, bsel)
    cn, ef = sc_gather(
        node2community, edge_features.reshape(n_edges),
        src_sel.reshape(bsel), eidx.reshape(bsel))

    ci_pad = jnp.full((1, bsel), -1, dtype=jnp.int32)
    ci_pad = lax.dynamic_update_slice(
        ci_pad, community_index.astype(jnp.int32).reshape(1, -1), (0, 0))

    def parts(w):
        return (w[:d], w[d:2 * d], w[2 * d:2 * d + 1], w[2 * d + 1:])

    wza, wzb, wze, wzt = parts(gru_Wz)
    wra, wrb, wre, wrt = parts(gru_Wr)
    wha, whb, whe, wht = parts(gru_Wh)

    nblk = 4
    blk = bsel // nblk
    col = pl.BlockSpec((blk, 1), lambda i: (i, 0))
    full = lambda shape: pl.BlockSpec(shape, lambda i: tuple(0 for _ in shape))
    smem = pl.BlockSpec(memory_space=pltpu.MemorySpace.SMEM)
    anyspace = pl.BlockSpec(memory_space=pl.ANY)

    pred = pl.pallas_call(
        _fused_body,
        grid=(nblk,),
        in_specs=[smem, smem, col, col, col, full((1, bsel)),
                  anyspace, anyspace]
                 + [full((1, d))] * 2
                 + [full((d, d)), full((d, d)), full((1, d)), full((d, d)),
                    full((d, d)), full((1, d))] * 3
                 + [full((d, d)), full((d, d)), full((1, d)),
                    full((1, d)), full((1, 1))],
        out_specs=col,
        out_shape=jax.ShapeDtypeStruct((bsel, 1), jnp.float32),
        scratch_shapes=[
            pltpu.VMEM((2, blk, d), jnp.float32),
            pltpu.VMEM((2, blk, d), jnp.float32),
            pltpu.VMEM((2, blk, d), jnp.float32),
            pltpu.SemaphoreType.DMA,
            pltpu.SemaphoreType.DMA,
        ],
    )(src_sel.reshape(bsel), y_node.reshape(bsel),
      t, ef.reshape(bsel, 1), cn.reshape(bsel, 1), ci_pad,
      memory, community_embeddings,
      time_w.reshape(1, d), time_b.reshape(1, d),
      wza, wzb, wze.reshape(1, d), wzt, gru_Uz, gru_bz.reshape(1, d),
      wra, wrb, wre.reshape(1, d), wrt, gru_Ur, gru_br.reshape(1, d),
      wha, whb, whe.reshape(1, d), wht, gru_Uh, gru_bh.reshape(1, d),
      pred_W1[:d], pred_W1[d:], pred_b1.reshape(1, d),
      pred_W2.reshape(1, d), pred_b2.reshape(1, 1))
    return pred
